# Initial kernel scaffold; baseline (speedup 1.0000x reference)
import jax
import jax.numpy as jnp
from jax.experimental import pallas as pl


def kernel(x, batch, params):
    raise NotImplementedError("write your pallas kernel here")


# speedup vs baseline: 1.6969x; 1.6969x over previous
import numpy as np
import jax
import jax.numpy as jnp
from jax import lax
from jax.experimental import pallas as pl
from jax.experimental.pallas import tpu as pltpu

N = 8192
H = 64
K = 20
R = 256
BIG = 1e30
BN_SCALE = 1.0 / np.sqrt(1.0 + 1e-5)

_INTERPRET = False


def _elu(v):
    return jnp.where(v > 0, v, jnp.exp(v) - 1.0)


def _bdot(a, b):
    return jnp.dot(a.astype(jnp.bfloat16), b.astype(jnp.bfloat16),
                   preferred_element_type=jnp.float32)


def _mlp2_kernel(x_ref, w1_ref, w2_ref, o_ref):
    h = _elu(_bdot(x_ref[...], w1_ref[...]))
    o_ref[...] = _elu(_bdot(h, w2_ref[...]))


def _heads_kernel(f_ref, wo1_ref, wo2_ref, wo3_ref, ws1_ref, ws2_ref, ws3_ref,
                  out_ref, split_ref):
    f = f_ref[...]
    o = _elu(_bdot(f, wo1_ref[...]))
    o = _elu(_bdot(o, wo2_ref[...]))
    out_ref[...] = _bdot(o, wo3_ref[...])
    s = _elu(_bdot(f, ws1_ref[...]))
    s = _elu(_bdot(s, ws2_ref[...]))
    split_ref[...] = _bdot(s, ws3_ref[...])


def _knn_agg_kernel(ft_ref, fr_ref, f_ref, bcol_ref, brow_ref,
                    wt_ref, gb_ref, o_ref, d_scr, macc_scr):
    ft = ft_ref[...]
    fr = fr_ref[...]
    f_full = f_ref[...]
    sqc = jnp.sum(ft * ft, axis=0, keepdims=True)
    sqr = jnp.sum(fr * fr, axis=1, keepdims=True)
    d = sqr + sqc - 2.0 * _bdot(fr, ft)
    bq = bcol_ref[...]
    bc = brow_ref[0:1, :]
    d = jnp.where(bq != bc, BIG, d)
    d_scr[...] = d
    macc_scr[...] = jnp.full((R, H), -BIG, jnp.float32)
    colid = lax.broadcasted_iota(jnp.int32, (R, N), 1)
    wt = wt_ref[...].astype(jnp.bfloat16)
    b = gb_ref[0:1, :]
    gamma = gb_ref[2:3, :]
    beta = gb_ref[4:5, :]

    def body(_, carry):
        dd = d_scr[...]
        m = jnp.min(dd, axis=1, keepdims=True)
        key = jnp.where(dd == m, colid, jnp.int32(N))
        amin = jnp.min(key, axis=1, keepdims=True)
        onehot = colid == amin
        xj = jnp.dot(onehot.astype(jnp.float32), f_full,
                     preferred_element_type=jnp.float32,
                     precision=lax.Precision.HIGHEST)
        u = jnp.concatenate([fr, xj - fr], axis=1)
        z = jnp.dot(u.astype(jnp.bfloat16), wt,
                    preferred_element_type=jnp.float32) + b
        msg = _elu(z) * (BN_SCALE * gamma) + beta
        valid = m < (BIG * 0.5)
        macc_scr[...] = jnp.where(valid,
                                  jnp.maximum(macc_scr[...], msg),
                                  macc_scr[...])
        d_scr[...] = jnp.where(onehot, BIG, dd)
        return carry

    lax.fori_loop(0, K, body, 0)
    o_ref[...] = macc_scr[...] + fr


def _full(shape):
    return pl.BlockSpec(shape, lambda i: (0, 0))


@jax.jit
def kernel(x, batch, params):
    batch = batch.astype(jnp.int32)
    batch_col = batch.reshape(N, 1)
    batch_row = jnp.broadcast_to(batch.reshape(1, N), (8, N))

    feats = pl.pallas_call(
        _mlp2_kernel,
        grid=(8,),
        in_specs=[pl.BlockSpec((N // 8, 8), lambda i: (i, 0)),
                  _full((8, 32)), _full((32, H))],
        out_specs=pl.BlockSpec((N // 8, H), lambda i: (i, 0)),
        out_shape=jax.ShapeDtypeStruct((N, H), jnp.float32),
        interpret=_INTERPRET,
    )(x, params["W_enc1"].T, params["W_enc2"].T)

    for i in range(4):
        wt = params["conv_W"][i].T
        gb = jnp.concatenate(
            [jnp.broadcast_to(params["conv_b"][i].reshape(1, H), (2, H)),
             jnp.broadcast_to(params["conv_gamma"][i].reshape(1, H), (2, H)),
             jnp.broadcast_to(params["conv_beta"][i].reshape(1, H), (4, H))],
            axis=0)
        ft = feats.T
        feats = pl.pallas_call(
            _knn_agg_kernel,
            grid=(N // R,),
            in_specs=[_full((H, N)),
                      pl.BlockSpec((R, H), lambda i: (i, 0)),
                      _full((N, H)),
                      pl.BlockSpec((R, 1), lambda i: (i, 0)),
                      _full((8, N)),
                      _full((2 * H, H)),
                      _full((8, H))],
            out_specs=pl.BlockSpec((R, H), lambda i: (i, 0)),
            out_shape=jax.ShapeDtypeStruct((N, H), jnp.float32),
            scratch_shapes=[pltpu.VMEM((R, N), jnp.float32),
                            pltpu.VMEM((R, H), jnp.float32)],
            interpret=_INTERPRET,
        )(ft, feats, feats, batch_col, batch_row, wt, gb)

    out, split_logit = pl.pallas_call(
        _heads_kernel,
        grid=(8,),
        in_specs=[pl.BlockSpec((N // 8, H), lambda i: (i, 0)),
                  _full((H, 64)), _full((64, 32)), _full((32, 8)),
                  _full((H, 64)), _full((64, 32)), _full((32, 1))],
        out_specs=[pl.BlockSpec((N // 8, 8), lambda i: (i, 0)),
                   pl.BlockSpec((N // 8, 1), lambda i: (i, 0))],
        out_shape=[jax.ShapeDtypeStruct((N, 8), jnp.float32),
                   jax.ShapeDtypeStruct((N, 1), jnp.float32)],
        interpret=_INTERPRET,
    )(feats, params["W_o1"].T, params["W_o2"].T, params["W_o3"].T,
      params["W_s1"].T, params["W_s2"].T, params["W_s3"].T)

    return (out, split_logit, batch)


# speedup vs baseline: 4.3976x; 2.5915x over previous
import numpy as np
import jax
import jax.numpy as jnp
from jax import lax
from jax.experimental import pallas as pl
from jax.experimental.pallas import tpu as pltpu

N = 8192
H = 64
K = 20
R = 256
C = 512
NCH = N // C
BIG = 1e30
BN_SCALE = 1.0 / np.sqrt(1.0 + 1e-5)

_INTERPRET = False


def _elu(v):
    return jnp.where(v > 0, v, jnp.exp(v) - 1.0)


def _bdot(a, b):
    return jnp.dot(a.astype(jnp.bfloat16), b.astype(jnp.bfloat16),
                   preferred_element_type=jnp.float32)


def _mlp2_kernel(x_ref, w1_ref, w2_ref, o_ref):
    h = _elu(_bdot(x_ref[...], w1_ref[...]))
    o_ref[...] = _elu(_bdot(h, w2_ref[...]))


def _heads_kernel(f_ref, wo1_ref, wo2_ref, wo3_ref, ws1_ref, ws2_ref, ws3_ref,
                  out_ref, split_ref):
    f = f_ref[...]
    o = _elu(_bdot(f, wo1_ref[...]))
    o = _elu(_bdot(o, wo2_ref[...]))
    out_ref[...] = _bdot(o, wo3_ref[...])
    s = _elu(_bdot(f, ws1_ref[...]))
    s = _elu(_bdot(s, ws2_ref[...]))
    split_ref[...] = _bdot(s, ws3_ref[...])


def _knn_agg_kernel(sb_ref, tlo_ref, thi_ref,
                    ft_ref, fr_ref, f3_ref, sqr_ref, sqc_ref,
                    bcol_ref, brow_ref, wt_ref, gb_ref,
                    o_ref, d_scr, cmin_scr, camin_scr, e_scr, macc_scr):
    i = pl.program_id(0)
    w0 = sb_ref[tlo_ref[i]]
    w1 = sb_ref[thi_ref[i] + 1]
    c0 = w0 // C
    c1 = (w1 + C - 1) // C

    fr = fr_ref[...]
    frb = fr.astype(jnp.bfloat16)
    sqr = sqr_ref[...]
    bq = bcol_ref[...]
    wt = wt_ref[...].astype(jnp.bfloat16)
    b = gb_ref[0:1, :]
    gamma = gb_ref[2:3, :]
    beta = gb_ref[4:5, :]

    for j in range(NCH):
        @pl.when(jnp.logical_and(j >= c0, j < c1))
        def _(j=j):
            ftc = ft_ref[:, j * C:(j + 1) * C].astype(jnp.bfloat16)
            mm = jnp.dot(frb, ftc, preferred_element_type=jnp.float32)
            dch = sqr + sqc_ref[0:1, j * C:(j + 1) * C] - 2.0 * mm
            bc = brow_ref[0:1, j * C:(j + 1) * C]
            d_scr[:, j * C:(j + 1) * C] = jnp.where(bq != bc, BIG, dch)

    macc_scr[...] = jnp.full((R, H), -BIG, jnp.float32)

    def body(_, carry):
        cmin_scr[...] = jnp.full((R, 128), BIG, jnp.float32)
        camin_scr[...] = jnp.full((R, 128), N, jnp.int32)
        for j in range(NCH):
            @pl.when(jnp.logical_and(j >= c0, j < c1))
            def _(j=j):
                dch = d_scr[:, j * C:(j + 1) * C]
                cm = jnp.min(dch, axis=1, keepdims=True)
                colid = lax.broadcasted_iota(jnp.int32, (R, C), 1) + j * C
                ckey = jnp.where(dch == cm, colid, jnp.int32(N))
                cmin_scr[:, j:j + 1] = cm
                camin_scr[:, j:j + 1] = jnp.min(ckey, axis=1, keepdims=True)
        cmins = cmin_scr[:, 0:NCH]
        m = jnp.min(cmins, axis=1, keepdims=True)
        key2 = jnp.where(cmins == m, camin_scr[:, 0:NCH], jnp.int32(N))
        amin = jnp.min(key2, axis=1, keepdims=True)

        e_scr[...] = jnp.zeros((R, 3 * H), jnp.float32)
        for j in range(NCH):
            @pl.when(jnp.logical_and(j >= c0, j < c1))
            def _(j=j):
                colid = lax.broadcasted_iota(jnp.int32, (R, C), 1) + j * C
                onehot = (colid == amin)
                f3c = f3_ref[j * C:(j + 1) * C, :]
                e_scr[...] += jnp.dot(onehot.astype(jnp.bfloat16), f3c,
                                      preferred_element_type=jnp.float32)
                dch = d_scr[:, j * C:(j + 1) * C]
                d_scr[:, j * C:(j + 1) * C] = jnp.where(onehot, BIG, dch)
        e_acc = e_scr[...]
        xj = (e_acc[:, 0:H] + e_acc[:, H:2 * H]) + e_acc[:, 2 * H:3 * H]

        u = jnp.concatenate([fr, xj - fr], axis=1)
        z = jnp.dot(u.astype(jnp.bfloat16), wt,
                    preferred_element_type=jnp.float32) + b
        msg = _elu(z) * (BN_SCALE * gamma) + beta
        valid = m < (BIG * 0.5)
        macc_scr[...] = jnp.where(valid,
                                  jnp.maximum(macc_scr[...], msg),
                                  macc_scr[...])
        return carry

    lax.fori_loop(0, K, body, 0)
    o_ref[...] = macc_scr[...] + fr


def _full(shape):
    return pl.BlockSpec(shape, lambda i: (0, 0))


@jax.jit
def kernel(x, batch, params):
    batch = batch.astype(jnp.int32)
    batch_col = batch.reshape(N, 1)
    batch_row = jnp.broadcast_to(batch.reshape(1, N), (8, N))
    sbounds = jnp.searchsorted(batch, jnp.arange(5, dtype=jnp.int32)
                               ).astype(jnp.int32)
    tile_ids = jnp.arange(N // R, dtype=jnp.int32)
    tlo = batch[tile_ids * R]
    thi = batch[tile_ids * R + (R - 1)]
    sbounds = jnp.pad(sbounds, (0, 3))

    feats = pl.pallas_call(
        _mlp2_kernel,
        grid=(8,),
        in_specs=[pl.BlockSpec((N // 8, 8), lambda i: (i, 0)),
                  _full((8, 32)), _full((32, H))],
        out_specs=pl.BlockSpec((N // 8, H), lambda i: (i, 0)),
        out_shape=jax.ShapeDtypeStruct((N, H), jnp.float32),
        interpret=_INTERPRET,
    )(x, params["W_enc1"].T, params["W_enc2"].T)

    for i in range(4):
        wt = params["conv_W"][i].T
        gb = jnp.concatenate(
            [jnp.broadcast_to(params["conv_b"][i].reshape(1, H), (2, H)),
             jnp.broadcast_to(params["conv_gamma"][i].reshape(1, H), (2, H)),
             jnp.broadcast_to(params["conv_beta"][i].reshape(1, H), (4, H))],
            axis=0)
        ft = feats.T
        f1 = feats.astype(jnp.bfloat16)
        r1 = feats - f1.astype(jnp.float32)
        f2 = r1.astype(jnp.bfloat16)
        f3 = (r1 - f2.astype(jnp.float32)).astype(jnp.bfloat16)
        f3cat = jnp.concatenate([f1, f2, f3], axis=1)
        sq = jnp.sum(feats * feats, axis=1)
        sq_col = sq.reshape(N, 1)
        sq_row = jnp.broadcast_to(sq.reshape(1, N), (8, N))

        feats = pl.pallas_call(
            _knn_agg_kernel,
            grid=(N // R,),
            in_specs=[pl.BlockSpec(memory_space=pltpu.SMEM),
                      pl.BlockSpec(memory_space=pltpu.SMEM),
                      pl.BlockSpec(memory_space=pltpu.SMEM),
                      _full((H, N)),
                      pl.BlockSpec((R, H), lambda i: (i, 0)),
                      _full((N, 3 * H)),
                      pl.BlockSpec((R, 1), lambda i: (i, 0)),
                      _full((8, N)),
                      pl.BlockSpec((R, 1), lambda i: (i, 0)),
                      _full((8, N)),
                      _full((2 * H, H)),
                      _full((8, H))],
            out_specs=pl.BlockSpec((R, H), lambda i: (i, 0)),
            out_shape=jax.ShapeDtypeStruct((N, H), jnp.float32),
            scratch_shapes=[pltpu.VMEM((R, N), jnp.float32),
                            pltpu.VMEM((R, 128), jnp.float32),
                            pltpu.VMEM((R, 128), jnp.int32),
                            pltpu.VMEM((R, 3 * H), jnp.float32),
                            pltpu.VMEM((R, H), jnp.float32)],
            interpret=_INTERPRET,
        )(sbounds, tlo, thi, ft, feats, f3cat, sq_col, sq_row,
          batch_col, batch_row, wt, gb)

    out, split_logit = pl.pallas_call(
        _heads_kernel,
        grid=(8,),
        in_specs=[pl.BlockSpec((N // 8, H), lambda i: (i, 0)),
                  _full((H, 64)), _full((64, 32)), _full((32, 8)),
                  _full((H, 64)), _full((64, 32)), _full((32, 1))],
        out_specs=[pl.BlockSpec((N // 8, 8), lambda i: (i, 0)),
                   pl.BlockSpec((N // 8, 1), lambda i: (i, 0))],
        out_shape=[jax.ShapeDtypeStruct((N, 8), jnp.float32),
                   jax.ShapeDtypeStruct((N, 1), jnp.float32)],
        interpret=_INTERPRET,
    )(feats, params["W_o1"].T, params["W_o2"].T, params["W_o3"].T,
      params["W_s1"].T, params["W_s2"].T, params["W_s3"].T)

    return (out, split_logit, batch)
